# baseline (speedup 1.0000x reference)
import functools

import jax
import jax.numpy as jnp
from jax import lax
from jax.experimental import pallas as pl
from jax.experimental.pallas import tpu as pltpu
from jax.experimental.pallas import tpu_sc as plsc

BATCH = 1024
INPUT_SIZE = 2048
OUTPUT_SIZE = 2048
N_INPUTS = 4
NUM_COMBOS = 16
LANES = 16

NUM_CORES = 2
NUM_SUBCORES = 16
NUM_WORKERS = NUM_CORES * NUM_SUBCORES
ROWS_PER_WORKER = BATCH // NUM_WORKERS
GROUPS_PER_SLAB = 8
SLAB = LANES * GROUPS_PER_SLAB
NUM_SLABS = OUTPUT_SIZE // SLAB

_MESH = plsc.VectorSubcoreMesh(core_axis_name="c", subcore_axis_name="s")


@functools.partial(
    pl.kernel,
    mesh=_MESH,
    compiler_params=pltpu.CompilerParams(
        use_tc_tiling_on_sc=False, needs_layout_passes=False),
    out_type=jax.ShapeDtypeStruct((BATCH // 8, NUM_SLABS, 8, SLAB),
                                  jnp.float32),
    scratch_types=[
        pltpu.VMEM((ROWS_PER_WORKER * INPUT_SIZE,), jnp.float32),
        pltpu.VMEM((OUTPUT_SIZE, NUM_COMBOS), jnp.float32),
        pltpu.VMEM((OUTPUT_SIZE, N_INPUTS), jnp.int32),
        pltpu.VMEM((ROWS_PER_WORKER // 8, 8, SLAB), jnp.float32),
    ],
)
def _lut_sc(x_hbm, tab_hbm, map_hbm, out_hbm, x_v, tab_v, map_v, ostage_v):
    wid = lax.axis_index("s") * NUM_CORES + lax.axis_index("c")
    rblk_base = wid * (ROWS_PER_WORKER // 8)
    pltpu.sync_copy(
        x_hbm.at[pl.ds(wid * ROWS_PER_WORKER * INPUT_SIZE,
                       ROWS_PER_WORKER * INPUT_SIZE)], x_v)
    pltpu.sync_copy(tab_hbm, tab_v)
    pltpu.sync_copy(map_hbm, map_v)

    iota = lax.iota(jnp.int32, LANES)

    def slab_body(gg, carry):
        cbase = gg * SLAB
        for gi in range(GROUPS_PER_SLAB):
            nbase = cbase + gi * LANES
            nodes = iota + nbase
            c = [plsc.load_gather(tab_v, [nodes, jnp.full((LANES,), s, jnp.int32)])
                 for s in range(NUM_COMBOS)]
            for maskb in (1, 2, 4, 8):
                for s in range(NUM_COMBOS):
                    if s & maskb:
                        c[s] = c[s] - c[s ^ maskb]
            midx = [plsc.load_gather(map_v, [nodes, jnp.full((LANES,), k, jnp.int32)])
                    for k in range(N_INPUTS)]
            gidx = [((midx[k] >> 7) << 10) | (midx[k] & (SLAB - 1))
                    for k in range(N_INPUTS)]

            @plsc.parallel_loop(0, ROWS_PER_WORKER, unroll=8)
            def row_body(r, c=c, gidx=gidx, gi=gi):
                rb = r >> 3
                rm = r & 7
                roff = pl.multiple_of((rb << 14) | (rm << 7), SLAB)
                x_row = x_v.at[pl.ds(roff, (INPUT_SIZE - SLAB) * 8 + SLAB)]
                m = [plsc.load_gather(x_row, [gidx[k]])
                     for k in range(N_INPUTS)]
                h = {s: c[s] for s in range(NUM_COMBOS)}
                for maskb, kbit in ((8, 3), (4, 2), (2, 1), (1, 0)):
                    h = {s: h[s] + m[kbit] * h[s | maskb]
                         for s in h if not s & maskb}
                ostage_v[rb, rm, pl.ds(gi * LANES, LANES)] = h[0]
        pltpu.sync_copy(
            ostage_v,
            out_hbm.at[pl.ds(rblk_base, ROWS_PER_WORKER // 8), gg],
        )
        return carry

    lax.fori_loop(0, NUM_SLABS, slab_body, 0)


def kernel(x, mapping, table):
    x4 = x.reshape(BATCH // 8, 8, INPUT_SIZE // SLAB, SLAB).transpose(0, 2, 1, 3)
    out4 = _lut_sc(x4.reshape(-1), table, mapping)
    return out4.transpose(0, 2, 1, 3).reshape(BATCH, OUTPUT_SIZE)


# speedup vs baseline: 1.3619x; 1.3619x over previous
import functools

import jax
import jax.numpy as jnp
from jax import lax
from jax.experimental import pallas as pl
from jax.experimental.pallas import tpu as pltpu
from jax.experimental.pallas import tpu_sc as plsc

BATCH = 1024
INPUT_SIZE = 2048
OUTPUT_SIZE = 2048
N_INPUTS = 4
NUM_COMBOS = 16
LANES = 16

NUM_CORES = 2
NUM_SUBCORES = 16
NUM_WORKERS = NUM_CORES * NUM_SUBCORES
ROWS_PER_WORKER = BATCH // NUM_WORKERS
GROUPS_PER_SLAB = 8
SLAB = LANES * GROUPS_PER_SLAB
NUM_SLABS = OUTPUT_SIZE // SLAB

_MESH = plsc.VectorSubcoreMesh(core_axis_name="c", subcore_axis_name="s")


@functools.partial(
    pl.kernel,
    mesh=_MESH,
    compiler_params=pltpu.CompilerParams(
        use_tc_tiling_on_sc=False, needs_layout_passes=False),
    out_type=jax.ShapeDtypeStruct((BATCH // 8, NUM_SLABS, 8, SLAB),
                                  jnp.float32),
    scratch_types=[
        pltpu.VMEM((ROWS_PER_WORKER * INPUT_SIZE,), jnp.float32),
        pltpu.VMEM((OUTPUT_SIZE, NUM_COMBOS), jnp.float32),
        pltpu.VMEM((OUTPUT_SIZE, N_INPUTS), jnp.int32),
        pltpu.VMEM((2, ROWS_PER_WORKER // 8, 8, SLAB), jnp.float32),
        pltpu.SemaphoreType.DMA,
    ],
)
def _lut_sc(x_hbm, tab_hbm, map_hbm, out_hbm, x_v, tab_v, map_v, ostage_v,
            osem):
    wid = lax.axis_index("s") * NUM_CORES + lax.axis_index("c")
    rblk_base = wid * (ROWS_PER_WORKER // 8)
    pltpu.sync_copy(
        x_hbm.at[pl.ds(wid * ROWS_PER_WORKER * INPUT_SIZE,
                       ROWS_PER_WORKER * INPUT_SIZE)], x_v)
    pltpu.sync_copy(tab_hbm, tab_v)
    pltpu.sync_copy(map_hbm, map_v)

    iota = lax.iota(jnp.int32, LANES)

    def slab_body(gg, carry):
        cbase = gg * SLAB
        obuf = ostage_v.at[gg & 1]
        for gi in range(GROUPS_PER_SLAB):
            nbase = cbase + gi * LANES
            nodes = iota + nbase
            c = [plsc.load_gather(tab_v, [nodes, jnp.full((LANES,), s, jnp.int32)])
                 for s in range(NUM_COMBOS)]
            for maskb in (1, 2, 4, 8):
                for s in range(NUM_COMBOS):
                    if s & maskb:
                        c[s] = c[s] - c[s ^ maskb]
            midx = [plsc.load_gather(map_v, [nodes, jnp.full((LANES,), k, jnp.int32)])
                    for k in range(N_INPUTS)]
            gidx = [((midx[k] >> 7) << 10) | (midx[k] & (SLAB - 1))
                    for k in range(N_INPUTS)]

            @plsc.parallel_loop(0, ROWS_PER_WORKER, unroll=4)
            def row_body(r, c=c, gidx=gidx, gi=gi):
                rb = r >> 3
                rm = r & 7
                roff = pl.multiple_of((rb << 14) | (rm << 7), SLAB)
                x_row = x_v.at[pl.ds(roff, (INPUT_SIZE - SLAB) * 8 + SLAB)]
                m = [plsc.load_gather(x_row, [gidx[k]])
                     for k in range(N_INPUTS)]
                h = {s: c[s] for s in range(NUM_COMBOS)}
                for maskb, kbit in ((8, 3), (4, 2), (2, 1), (1, 0)):
                    h = {s: h[s] + m[kbit] * h[s | maskb]
                         for s in h if not s & maskb}
                obuf[rb, rm, pl.ds(gi * LANES, LANES)] = h[0]
        dst = out_hbm.at[pl.ds(rblk_base, ROWS_PER_WORKER // 8), gg]

        @pl.when(gg > 0)
        def _():
            pltpu.make_async_copy(obuf, dst, osem).wait()

        pltpu.make_async_copy(obuf, dst, osem).start()
        return carry

    lax.fori_loop(0, NUM_SLABS, slab_body, 0)
    pltpu.make_async_copy(
        ostage_v.at[(NUM_SLABS - 1) & 1],
        out_hbm.at[pl.ds(rblk_base, ROWS_PER_WORKER // 8), NUM_SLABS - 1],
        osem,
    ).wait()


def kernel(x, mapping, table):
    x4 = x.reshape(BATCH // 8, 8, INPUT_SIZE // SLAB, SLAB).transpose(0, 2, 1, 3)
    out4 = _lut_sc(x4.reshape(-1), table, mapping)
    return out4.transpose(0, 2, 1, 3).reshape(BATCH, OUTPUT_SIZE)


# speedup vs baseline: 1.3907x; 1.0211x over previous
import functools

import jax
import jax.numpy as jnp
from jax import lax
from jax.experimental import pallas as pl
from jax.experimental.pallas import tpu as pltpu
from jax.experimental.pallas import tpu_sc as plsc

BATCH = 1024
INPUT_SIZE = 2048
OUTPUT_SIZE = 2048
N_INPUTS = 4
NUM_COMBOS = 16
LANES = 16

NUM_CORES = 2
NUM_SUBCORES = 16
NUM_WORKERS = NUM_CORES * NUM_SUBCORES
ROWS_PER_WORKER = BATCH // NUM_WORKERS
GROUPS_PER_SLAB = 8
SLAB = LANES * GROUPS_PER_SLAB
NUM_SLABS = OUTPUT_SIZE // SLAB

_MESH = plsc.VectorSubcoreMesh(core_axis_name="c", subcore_axis_name="s")


@functools.partial(
    pl.kernel,
    mesh=_MESH,
    compiler_params=pltpu.CompilerParams(
        use_tc_tiling_on_sc=False, needs_layout_passes=False),
    out_type=jax.ShapeDtypeStruct((BATCH // 8, NUM_SLABS, 8, SLAB),
                                  jnp.float32),
    scratch_types=[
        pltpu.VMEM((ROWS_PER_WORKER * INPUT_SIZE,), jnp.float32),
        pltpu.VMEM((OUTPUT_SIZE, NUM_COMBOS), jnp.float32),
        pltpu.VMEM((OUTPUT_SIZE, N_INPUTS), jnp.int32),
        pltpu.VMEM((2, ROWS_PER_WORKER // 8, 8, SLAB), jnp.float32),
        pltpu.SemaphoreType.DMA,
        pltpu.SemaphoreType.DMA,
    ],
)
def _lut_sc(x_hbm, tab_hbm, map_hbm, out_hbm, x_v, tab_v, map_v, ostage_v,
            osem, isem):
    wid = lax.axis_index("s") * NUM_CORES + lax.axis_index("c")
    rblk_base = wid * (ROWS_PER_WORKER // 8)
    in_x = pltpu.make_async_copy(
        x_hbm.at[pl.ds(wid * ROWS_PER_WORKER * INPUT_SIZE,
                       ROWS_PER_WORKER * INPUT_SIZE)], x_v, isem)
    in_tab = pltpu.make_async_copy(tab_hbm, tab_v, isem)
    in_map = pltpu.make_async_copy(map_hbm, map_v, isem)
    in_x.start()
    in_tab.start()
    in_map.start()
    in_x.wait()
    in_tab.wait()
    in_map.wait()

    iota = lax.iota(jnp.int32, LANES)

    def slab_body(gg, carry):
        cbase = gg * SLAB
        obuf = ostage_v.at[gg & 1]
        for gi in range(GROUPS_PER_SLAB):
            nbase = cbase + gi * LANES
            nodes = iota + nbase
            c = [plsc.load_gather(tab_v, [nodes, jnp.full((LANES,), s, jnp.int32)])
                 for s in range(NUM_COMBOS)]
            for maskb in (1, 2, 4, 8):
                for s in range(NUM_COMBOS):
                    if s & maskb:
                        c[s] = c[s] - c[s ^ maskb]
            midx = [plsc.load_gather(map_v, [nodes, jnp.full((LANES,), k, jnp.int32)])
                    for k in range(N_INPUTS)]
            gidx = [((midx[k] >> 7) << 10) | (midx[k] & (SLAB - 1))
                    for k in range(N_INPUTS)]

            @plsc.parallel_loop(0, ROWS_PER_WORKER, unroll=4)
            def row_body(r, c=c, gidx=gidx, gi=gi):
                rb = r >> 3
                rm = r & 7
                roff = pl.multiple_of((rb << 14) | (rm << 7), SLAB)
                x_row = x_v.at[pl.ds(roff, (INPUT_SIZE - SLAB) * 8 + SLAB)]
                m = [plsc.load_gather(x_row, [gidx[k]])
                     for k in range(N_INPUTS)]
                h = {s: c[s] for s in range(NUM_COMBOS)}
                for maskb, kbit in ((8, 3), (4, 2), (2, 1), (1, 0)):
                    h = {s: h[s] + m[kbit] * h[s | maskb]
                         for s in h if not s & maskb}
                obuf[rb, rm, pl.ds(gi * LANES, LANES)] = h[0]
        dst = out_hbm.at[pl.ds(rblk_base, ROWS_PER_WORKER // 8), gg]

        @pl.when(gg > 0)
        def _():
            pltpu.make_async_copy(obuf, dst, osem).wait()

        pltpu.make_async_copy(obuf, dst, osem).start()
        return carry

    lax.fori_loop(0, NUM_SLABS, slab_body, 0)
    pltpu.make_async_copy(
        ostage_v.at[(NUM_SLABS - 1) & 1],
        out_hbm.at[pl.ds(rblk_base, ROWS_PER_WORKER // 8), NUM_SLABS - 1],
        osem,
    ).wait()


def kernel(x, mapping, table):
    x4 = x.reshape(BATCH // 8, 8, INPUT_SIZE // SLAB, SLAB).transpose(0, 2, 1, 3)
    out4 = _lut_sc(x4.reshape(-1), table, mapping)
    return out4.transpose(0, 2, 1, 3).reshape(BATCH, OUTPUT_SIZE)
